# baseline (speedup 1.0000x reference)
import math

import jax
import jax.numpy as jnp
from jax import lax
from jax.experimental import pallas as pl
from jax.experimental.pallas import tpu as pltpu
from jax.experimental.pallas import tpu_sc as plsc

B = 4096
NCAT = 26
L = 200
EMB = 64
ATTN = 128
HID = 512
TOPK = 30

R = 128
GRID = B // R
NC, NS = 2, 16
NW = NC * NS

NSLICE = 4
BS = B // NSLICE
SEQ_PER_W = BS * L // NW
SEQ_N128 = SEQ_PER_W // 128
CAT_TOTAL = B * NCAT + B
CAT_PER_W = CAT_TOTAL // NW


def _sc_gather_kernel(with_cat, *refs):
    if with_cat:
        (seq_emb, seq_idx, cat_emb, cat_idx, out_seq, out_cat,
         idx_seq_v, idx_cat_v, rows_v, sem) = refs
    else:
        seq_emb, seq_idx, out_seq, idx_seq_v, rows_v, sem = refs
    wid = lax.axis_index("s") * NC + lax.axis_index("c")
    pltpu.sync_copy(seq_idx.at[wid], idx_seq_v)

    seq_base = wid * SEQ_PER_W

    @pl.loop(0, 5)
    def _seq_chunk(j):
        cps = [
            pltpu.async_copy(
                seq_emb.at[idx_seq_v.at[j * 10 + b]],
                rows_v.at[pl.ds(b * 128, 128)],
                sem,
            )
            for b in range(10)
        ]
        for cp in cps:
            cp.wait()
        pltpu.sync_copy(rows_v.at[pl.ds(0, 1280)],
                        out_seq.at[pl.ds(seq_base + j * 1280, 1280)])

    if with_cat:
        pltpu.sync_copy(cat_idx.at[wid], idx_cat_v)
        cat_base = wid * CAT_PER_W

        @pl.loop(0, 3)
        def _cat_chunk(g):
            cps = [
                pltpu.async_copy(
                    cat_emb.at[idx_cat_v.at[g * 9 + b]],
                    rows_v.at[pl.ds(b * 128, 128)],
                    sem,
                )
                for b in range(9)
            ]
            for cp in cps:
                cp.wait()
            pltpu.sync_copy(rows_v.at[pl.ds(0, 1152)],
                            out_cat.at[pl.ds(cat_base + g * 1152, 1152)])


def _sc_gather(seq_emb, seq_idx3, cat_emb=None, cat_idx3=None):
    with_cat = cat_emb is not None
    mesh = plsc.VectorSubcoreMesh(core_axis_name="c", subcore_axis_name="s")
    out_type = [jax.ShapeDtypeStruct((BS * L, EMB), jnp.float32)]
    scratch = [pltpu.VMEM((SEQ_N128, 128), jnp.int32)]
    args = [seq_emb, seq_idx3]
    if with_cat:
        out_type.append(jax.ShapeDtypeStruct((CAT_TOTAL, EMB), jnp.float32))
        scratch.append(pltpu.VMEM((27, 128), jnp.int32))
        args += [cat_emb, cat_idx3]
        args = [seq_emb, seq_idx3, cat_emb, cat_idx3]
    scratch += [pltpu.VMEM((1280, EMB), jnp.float32), pltpu.SemaphoreType.DMA]
    f = pl.kernel(
        lambda *r: _sc_gather_kernel(with_cat, *r),
        out_type=out_type,
        mesh=mesh,
        scratch_types=scratch,
        compiler_params=pltpu.CompilerParams(use_tc_tiling_on_sc=False),
    )
    return f(*args)


def _qnn(x, w1, b1, w2, b2, wo, bo, g, be):
    quad = (x @ w1 + b1) * (x @ w2 + b2)
    y = x + quad @ wo + bo
    mu = jnp.mean(y, axis=-1, keepdims=True)
    var = jnp.mean((y - mu) ** 2, axis=-1, keepdims=True)
    return g * (y - mu) / jnp.sqrt(var + 1e-5) + be


def _tc_body(seqflat_ref, catp_ref, tgt_ref, nums_ref,
             q_w_ref, k_w_ref, v_w_ref, ao_w_ref, ao_b_ref,
             cp_w_ref, cp_b_ref, num_w_ref, num_b_ref,
             mlp_w_ref, mlp_b_ref,
             a_w1_ref, a_b1_ref, a_w2_ref, a_b2_ref, a_wo_ref, a_bo_ref,
             a_g_ref, a_be_ref,
             b_w1_ref, b_b1_ref, b_w2_ref, b_b2_ref, b_wo_ref, b_bo_ref,
             b_g_ref, b_be_ref,
             out_w_ref, out_b_ref, out_ref):
    tgt = tgt_ref[...]
    q = tgt @ q_w_ref[...]
    q2 = lax.dot_general(q, k_w_ref[...], (((1,), (1,)), ((), ())))
    q2 = q2 * (1.0 / math.sqrt(float(ATTN)))

    seq3 = seqflat_ref[...].reshape(R, L, EMB)
    LC = 40
    scores = jnp.concatenate(
        [jnp.sum(seq3[:, c:c + LC, :] * q2[:, None, :], axis=-1)
         for c in range(0, L, LC)], axis=1)

    bits = lax.bitcast_convert_type(scores, jnp.int32)
    keys = jnp.where(bits >= 0, bits, bits ^ jnp.int32(0x7FFFFFFF))

    lo0 = jnp.full((R, 1), jnp.iinfo(jnp.int32).min, jnp.int32)
    hi0 = jnp.full((R, 1), jnp.iinfo(jnp.int32).max, jnp.int32)

    def bs(t, carry):
        lo, hi = carry
        mid = (lo >> 1) + (hi >> 1) + (lo & hi & 1) + ((lo ^ hi) & 1)
        cnt = jnp.sum(jnp.where(keys >= mid, 1.0, 0.0), axis=1,
                      keepdims=True)
        pred = cnt >= float(TOPK)
        return jnp.where(pred, mid, lo), jnp.where(pred, hi, mid - 1)

    thr, _ = lax.fori_loop(0, 32, bs, (lo0, hi0))

    gt = keys > thr
    eq = keys == thr
    c1 = jnp.sum(jnp.where(gt, 1.0, 0.0), axis=1, keepdims=True)
    ri = lax.broadcasted_iota(jnp.int32, (L, L), 0)
    ci = lax.broadcasted_iota(jnp.int32, (L, L), 1)
    lower_tri = jnp.where(ri <= ci, 1.0, 0.0)
    csum = jnp.where(eq, 1.0, 0.0) @ lower_tri
    sel = gt | (eq & (csum <= (float(TOPK) - c1)))

    masked = jnp.where(sel, scores, -jnp.inf)
    mx = jnp.max(masked, axis=1, keepdims=True)
    ex = jnp.exp(masked - mx)
    wgt = ex / jnp.sum(ex, axis=1, keepdims=True)

    wsum = jnp.zeros((R, EMB), jnp.float32)
    for c in range(0, L, LC):
        wsum = wsum + jnp.sum(seq3[:, c:c + LC, :]
                              * wgt[:, c:c + LC, None], axis=1)
    vp = lax.dot_general(wsum, v_w_ref[...], (((1,), (0,)), ((), ())))
    interest = vp @ ao_w_ref[...] + ao_b_ref[...]

    cat_pool = catp_ref[...] @ cp_w_ref[...] + cp_b_ref[...]
    num_e = nums_ref[...] @ num_w_ref[...] + num_b_ref[...]

    mlp_w = mlp_w_ref[...]
    h = (tgt @ mlp_w[0:EMB]
         + interest @ mlp_w[EMB:2 * EMB]
         + cat_pool @ mlp_w[2 * EMB:3 * EMB]
         + num_e @ mlp_w[3 * EMB:4 * EMB]
         + mlp_b_ref[...])
    h = jnp.maximum(h, 0.0)

    h = _qnn(h, a_w1_ref[...], a_b1_ref[...], a_w2_ref[...], a_b2_ref[...],
             a_wo_ref[...], a_bo_ref[...], a_g_ref[...], a_be_ref[...])
    h = _qnn(h, b_w1_ref[...], b_b1_ref[...], b_w2_ref[...], b_b2_ref[...],
             b_wo_ref[...], b_bo_ref[...], b_g_ref[...], b_be_ref[...])

    out_ref[...] = h @ out_w_ref[...] + out_b_ref[...]


def _row_block(shape, grid):
    nd = len(shape)
    return pl.BlockSpec((shape[0] // grid,) + shape[1:],
                        lambda i: (i,) + (0,) * (nd - 1))


def _whole(shape):
    nd = len(shape)
    return pl.BlockSpec(shape, lambda i: (0,) * nd)


def _fused_tc(seqflat, catp, tgt_e, nums, *weights, interpret=False):
    bs = tgt_e.shape[0]
    grid = bs // R
    in_specs = [
        _row_block((bs * L, EMB), grid),
        _row_block((bs, NCAT * EMB), grid),
        _row_block((bs, EMB), grid),
        _row_block((bs, nums.shape[1]), grid),
    ] + [_whole(w.shape) for w in weights]
    f = pl.pallas_call(
        _tc_body,
        grid=(grid,),
        in_specs=in_specs,
        out_specs=_row_block((bs, 1), grid),
        out_shape=jax.ShapeDtypeStruct((bs, 1), jnp.float32),
        compiler_params=pltpu.CompilerParams(
            dimension_semantics=("arbitrary",),
            vmem_limit_bytes=100 * 1024 * 1024,
        ),
        interpret=interpret,
    )
    return f(seqflat, catp, tgt_e, nums, *weights)


def kernel(cats, nums, seqs, tgt_ids, cat_emb, seq_emb, num_w, num_b, q_w,
           k_w, v_w, ao_w, ao_b, cp_w, cp_b, mlp_w, mlp_b,
           a_w1, a_b1, a_w2, a_b2, a_wo, a_bo, a_g, a_be,
           b_w1, b_b1, b_w2, b_b2, b_wo, b_bo, b_g, b_be,
           out_w, out_b):
    cat_idx3 = jnp.concatenate(
        [cats.astype(jnp.int32).reshape(-1), tgt_ids.astype(jnp.int32)]
    ).reshape(NW, CAT_PER_W // 128, 128)

    seq_parts = []
    out_cat = None
    for s in range(NSLICE):
        seq_idx3 = (seqs[s * BS:(s + 1) * BS].astype(jnp.int32)
                    .reshape(NW, SEQ_N128, 128))
        if s == 0:
            part, out_cat = _sc_gather(seq_emb, seq_idx3, cat_emb, cat_idx3)
        else:
            (part,) = _sc_gather(seq_emb, seq_idx3)
        seq_parts.append(part)
    catp = out_cat[:B * NCAT].reshape(B, NCAT * EMB)
    tgt_e = out_cat[B * NCAT:]

    weights = (q_w, k_w, v_w, ao_w, ao_b.reshape(1, EMB),
               cp_w, cp_b.reshape(1, EMB), num_w, num_b.reshape(1, EMB),
               mlp_w, mlp_b.reshape(1, HID),
               a_w1, a_b1.reshape(1, HID), a_w2, a_b2.reshape(1, HID),
               a_wo, a_bo.reshape(1, HID), a_g.reshape(1, HID),
               a_be.reshape(1, HID),
               b_w1, b_b1.reshape(1, HID), b_w2, b_b2.reshape(1, HID),
               b_wo, b_bo.reshape(1, HID), b_g.reshape(1, HID),
               b_be.reshape(1, HID),
               out_w, out_b.reshape(1, 1))
    logits = []
    for s in range(NSLICE):
        sl = slice(s * BS, (s + 1) * BS)
        logits.append(_fused_tc(seq_parts[s], catp[sl], tgt_e[sl], nums[sl],
                                *weights))
    return jnp.concatenate(logits, axis=0)[:, 0]


# speedup vs baseline: 1.0341x; 1.0341x over previous
import math

import jax
import jax.numpy as jnp
from jax import lax
from jax.experimental import pallas as pl
from jax.experimental.pallas import tpu as pltpu
from jax.experimental.pallas import tpu_sc as plsc

B = 4096
NCAT = 26
L = 200
EMB = 64
ATTN = 128
HID = 512
TOPK = 30

R = 128
GRID = B // R
NC, NS = 2, 16
NW = NC * NS

NSLICE = 1
BS = B // NSLICE
SEQ_PER_W = BS * L // NW
SEQ_N128 = SEQ_PER_W // 128
CAT_TOTAL = B * NCAT + B
CAT_PER_W = CAT_TOTAL // NW


def _sc_gather_kernel(with_cat, *refs):
    if with_cat:
        (seq_emb, seq_idx, cat_emb, cat_idx, out_seq, out_cat,
         idx_seq_v, idx_cat_v, rows_v, sem) = refs
    else:
        seq_emb, seq_idx, out_seq, idx_seq_v, rows_v, sem = refs
    wid = lax.axis_index("s") * NC + lax.axis_index("c")
    pltpu.sync_copy(seq_idx.at[wid], idx_seq_v)

    seq_base = wid * SEQ_PER_W

    @pl.loop(0, SEQ_PER_W // 1280)
    def _seq_chunk(j):
        cps = [
            pltpu.async_copy(
                seq_emb.at[idx_seq_v.at[j * 10 + b]],
                rows_v.at[pl.ds(b * 128, 128)],
                sem,
            )
            for b in range(10)
        ]
        for cp in cps:
            cp.wait()
        pltpu.sync_copy(rows_v.at[pl.ds(0, 1280)],
                        out_seq.at[pl.ds(seq_base + j * 1280, 1280)])

    if with_cat:
        pltpu.sync_copy(cat_idx.at[wid], idx_cat_v)
        cat_base = wid * CAT_PER_W

        @pl.loop(0, 3)
        def _cat_chunk(g):
            cps = [
                pltpu.async_copy(
                    cat_emb.at[idx_cat_v.at[g * 9 + b]],
                    rows_v.at[pl.ds(b * 128, 128)],
                    sem,
                )
                for b in range(9)
            ]
            for cp in cps:
                cp.wait()
            pltpu.sync_copy(rows_v.at[pl.ds(0, 1152)],
                            out_cat.at[pl.ds(cat_base + g * 1152, 1152)])


def _sc_gather(seq_emb, seq_idx3, cat_emb=None, cat_idx3=None):
    with_cat = cat_emb is not None
    mesh = plsc.VectorSubcoreMesh(core_axis_name="c", subcore_axis_name="s")
    out_type = [jax.ShapeDtypeStruct((BS * L, EMB), jnp.float32)]
    scratch = [pltpu.VMEM((SEQ_N128, 128), jnp.int32)]
    args = [seq_emb, seq_idx3]
    if with_cat:
        out_type.append(jax.ShapeDtypeStruct((CAT_TOTAL, EMB), jnp.float32))
        scratch.append(pltpu.VMEM((27, 128), jnp.int32))
        args += [cat_emb, cat_idx3]
        args = [seq_emb, seq_idx3, cat_emb, cat_idx3]
    scratch += [pltpu.VMEM((1280, EMB), jnp.float32), pltpu.SemaphoreType.DMA]
    f = pl.kernel(
        lambda *r: _sc_gather_kernel(with_cat, *r),
        out_type=out_type,
        mesh=mesh,
        scratch_types=scratch,
        compiler_params=pltpu.CompilerParams(use_tc_tiling_on_sc=False),
    )
    return f(*args)


def _qnn(x, w1, b1, w2, b2, wo, bo, g, be):
    quad = (x @ w1 + b1) * (x @ w2 + b2)
    y = x + quad @ wo + bo
    mu = jnp.mean(y, axis=-1, keepdims=True)
    var = jnp.mean((y - mu) ** 2, axis=-1, keepdims=True)
    return g * (y - mu) / jnp.sqrt(var + 1e-5) + be


def _tc_body(seqflat_ref, catp_ref, tgt_ref, nums_ref,
             q_w_ref, k_w_ref, v_w_ref, ao_w_ref, ao_b_ref,
             cp_w_ref, cp_b_ref, num_w_ref, num_b_ref,
             mlp_w_ref, mlp_b_ref,
             a_w1_ref, a_b1_ref, a_w2_ref, a_b2_ref, a_wo_ref, a_bo_ref,
             a_g_ref, a_be_ref,
             b_w1_ref, b_b1_ref, b_w2_ref, b_b2_ref, b_wo_ref, b_bo_ref,
             b_g_ref, b_be_ref,
             out_w_ref, out_b_ref, out_ref):
    tgt = tgt_ref[...]
    q = tgt @ q_w_ref[...]
    q2 = lax.dot_general(q, k_w_ref[...], (((1,), (1,)), ((), ())))
    q2 = q2 * (1.0 / math.sqrt(float(ATTN)))

    seq3 = seqflat_ref[...].reshape(R, L, EMB)
    LC = 40
    scores = jnp.concatenate(
        [jnp.sum(seq3[:, c:c + LC, :] * q2[:, None, :], axis=-1)
         for c in range(0, L, LC)], axis=1)

    bits = lax.bitcast_convert_type(scores, jnp.int32)
    keys = jnp.where(bits >= 0, bits, bits ^ jnp.int32(0x7FFFFFFF))

    lo0 = jnp.full((R, 1), jnp.iinfo(jnp.int32).min, jnp.int32)
    hi0 = jnp.full((R, 1), jnp.iinfo(jnp.int32).max, jnp.int32)

    def bs(t, carry):
        lo, hi = carry
        mid = (lo >> 1) + (hi >> 1) + (lo & hi & 1) + ((lo ^ hi) & 1)
        cnt = jnp.sum(jnp.where(keys >= mid, 1.0, 0.0), axis=1,
                      keepdims=True)
        pred = cnt >= float(TOPK)
        return jnp.where(pred, mid, lo), jnp.where(pred, hi, mid - 1)

    thr, _ = lax.fori_loop(0, 32, bs, (lo0, hi0))

    gt = keys > thr
    eq = keys == thr
    c1 = jnp.sum(jnp.where(gt, 1.0, 0.0), axis=1, keepdims=True)
    ri = lax.broadcasted_iota(jnp.int32, (L, L), 0)
    ci = lax.broadcasted_iota(jnp.int32, (L, L), 1)
    lower_tri = jnp.where(ri <= ci, 1.0, 0.0)
    csum = jnp.where(eq, 1.0, 0.0) @ lower_tri
    sel = gt | (eq & (csum <= (float(TOPK) - c1)))

    masked = jnp.where(sel, scores, -jnp.inf)
    mx = jnp.max(masked, axis=1, keepdims=True)
    ex = jnp.exp(masked - mx)
    wgt = ex / jnp.sum(ex, axis=1, keepdims=True)

    wsum = jnp.zeros((R, EMB), jnp.float32)
    for c in range(0, L, LC):
        wsum = wsum + jnp.sum(seq3[:, c:c + LC, :]
                              * wgt[:, c:c + LC, None], axis=1)
    vp = lax.dot_general(wsum, v_w_ref[...], (((1,), (0,)), ((), ())))
    interest = vp @ ao_w_ref[...] + ao_b_ref[...]

    cat_pool = catp_ref[...] @ cp_w_ref[...] + cp_b_ref[...]
    num_e = nums_ref[...] @ num_w_ref[...] + num_b_ref[...]

    mlp_w = mlp_w_ref[...]
    h = (tgt @ mlp_w[0:EMB]
         + interest @ mlp_w[EMB:2 * EMB]
         + cat_pool @ mlp_w[2 * EMB:3 * EMB]
         + num_e @ mlp_w[3 * EMB:4 * EMB]
         + mlp_b_ref[...])
    h = jnp.maximum(h, 0.0)

    h = _qnn(h, a_w1_ref[...], a_b1_ref[...], a_w2_ref[...], a_b2_ref[...],
             a_wo_ref[...], a_bo_ref[...], a_g_ref[...], a_be_ref[...])
    h = _qnn(h, b_w1_ref[...], b_b1_ref[...], b_w2_ref[...], b_b2_ref[...],
             b_wo_ref[...], b_bo_ref[...], b_g_ref[...], b_be_ref[...])

    out_ref[...] = h @ out_w_ref[...] + out_b_ref[...]


def _row_block(shape, grid):
    nd = len(shape)
    return pl.BlockSpec((shape[0] // grid,) + shape[1:],
                        lambda i: (i,) + (0,) * (nd - 1))


def _whole(shape):
    nd = len(shape)
    return pl.BlockSpec(shape, lambda i: (0,) * nd)


def _fused_tc(seqflat, catp, tgt_e, nums, *weights, interpret=False):
    bs = tgt_e.shape[0]
    grid = bs // R
    in_specs = [
        _row_block((bs * L, EMB), grid),
        _row_block((bs, NCAT * EMB), grid),
        _row_block((bs, EMB), grid),
        _row_block((bs, nums.shape[1]), grid),
    ] + [_whole(w.shape) for w in weights]
    f = pl.pallas_call(
        _tc_body,
        grid=(grid,),
        in_specs=in_specs,
        out_specs=_row_block((bs, 1), grid),
        out_shape=jax.ShapeDtypeStruct((bs, 1), jnp.float32),
        compiler_params=pltpu.CompilerParams(
            dimension_semantics=("arbitrary",),
            vmem_limit_bytes=100 * 1024 * 1024,
        ),
        interpret=interpret,
    )
    return f(seqflat, catp, tgt_e, nums, *weights)


def kernel(cats, nums, seqs, tgt_ids, cat_emb, seq_emb, num_w, num_b, q_w,
           k_w, v_w, ao_w, ao_b, cp_w, cp_b, mlp_w, mlp_b,
           a_w1, a_b1, a_w2, a_b2, a_wo, a_bo, a_g, a_be,
           b_w1, b_b1, b_w2, b_b2, b_wo, b_bo, b_g, b_be,
           out_w, out_b):
    cat_idx3 = jnp.concatenate(
        [cats.astype(jnp.int32).reshape(-1), tgt_ids.astype(jnp.int32)]
    ).reshape(NW, CAT_PER_W // 128, 128)

    seq_parts = []
    out_cat = None
    for s in range(NSLICE):
        seq_idx3 = (seqs[s * BS:(s + 1) * BS].astype(jnp.int32)
                    .reshape(NW, SEQ_N128, 128))
        if s == 0:
            part, out_cat = _sc_gather(seq_emb, seq_idx3, cat_emb, cat_idx3)
        else:
            (part,) = _sc_gather(seq_emb, seq_idx3)
        seq_parts.append(part)
    catp = out_cat[:B * NCAT].reshape(B, NCAT * EMB)
    tgt_e = out_cat[B * NCAT:]

    weights = (q_w, k_w, v_w, ao_w, ao_b.reshape(1, EMB),
               cp_w, cp_b.reshape(1, EMB), num_w, num_b.reshape(1, EMB),
               mlp_w, mlp_b.reshape(1, HID),
               a_w1, a_b1.reshape(1, HID), a_w2, a_b2.reshape(1, HID),
               a_wo, a_bo.reshape(1, HID), a_g.reshape(1, HID),
               a_be.reshape(1, HID),
               b_w1, b_b1.reshape(1, HID), b_w2, b_b2.reshape(1, HID),
               b_wo, b_bo.reshape(1, HID), b_g.reshape(1, HID),
               b_be.reshape(1, HID),
               out_w, out_b.reshape(1, 1))
    logits = []
    for s in range(NSLICE):
        sl = slice(s * BS, (s + 1) * BS)
        logits.append(_fused_tc(seq_parts[s], catp[sl], tgt_e[sl], nums[sl],
                                *weights))
    return jnp.concatenate(logits, axis=0)[:, 0]
